# Initial kernel scaffold; baseline (speedup 1.0000x reference)
import jax
import jax.numpy as jnp
from jax.experimental import pallas as pl


def kernel(A_a, X_a, Wr, br, W1, b1, W2, b2):
    raise NotImplementedError("write your pallas kernel here")


# speedup vs baseline: 2.6484x; 2.6484x over previous
import functools

import jax
import jax.numpy as jnp
from jax import lax
from jax.experimental import pallas as pl
from jax.experimental.pallas import tpu as pltpu
from jax.experimental.pallas import tpu_sc as plsc

_N = 10000
_D = 128
_E = 320000

_NC = 2
_NS = 16
_NW = _NC * _NS

_C = 128
_EPT = 10240
_NCHUNK = _EPT // _C
_EPAD = _EPT * _NW
_RPT = 632
_NROWS = _RPT * _NS


def _mm_bias_kernel(x_ref, w_ref, b_ref, o_ref):
    o_ref[...] = (
        jnp.dot(x_ref[...], w_ref[...], preferred_element_type=jnp.float32)
        + b_ref[...]
    )


def _mm_bias(x, w, b2d):
    return pl.pallas_call(
        _mm_bias_kernel,
        out_shape=jax.ShapeDtypeStruct((x.shape[0], w.shape[1]), jnp.float32),
    )(x, w, b2d)


def _merge_relu_mm_kernel(p0_ref, p1_ref, w_ref, b_ref, o_ref):
    x = jnp.maximum(p0_ref[...] + p1_ref[...], 0.0)
    o_ref[...] = (
        jnp.dot(x, w_ref[...], preferred_element_type=jnp.float32) + b_ref[...]
    )


def _merge_relu_mm(p0, p1, w, b2d):
    return pl.pallas_call(
        _merge_relu_mm_kernel,
        out_shape=jax.ShapeDtypeStruct((p0.shape[0], w.shape[1]), jnp.float32),
    )(p0, p1, w, b2d)


def _add_kernel(a_ref, b_ref, o_ref):
    o_ref[...] = a_ref[...] + b_ref[...]


def _merge_add(a, b):
    return pl.pallas_call(
        _add_kernel,
        out_shape=jax.ShapeDtypeStruct(a.shape, jnp.float32),
    )(a, b)


_mesh = plsc.VectorSubcoreMesh(core_axis_name="c", subcore_axis_name="s")


@functools.partial(
    pl.kernel,
    out_type=jax.ShapeDtypeStruct((_NC * _NROWS, _D), jnp.float32),
    mesh=_mesh,
    scratch_types=[
        pltpu.VMEM((_C,), jnp.int32),
        pltpu.VMEM((_C,), jnp.int32),
        pltpu.VMEM((_C, _D), jnp.float32),
        pltpu.VMEM_SHARED((_NROWS, _D), jnp.float32),
        pltpu.SemaphoreType.DMA,
    ],
)
def _aggregate(h_hbm, src_hbm, dst_hbm, z_hbm, out_hbm, sidx, didx, rows, acc, sem):
    cid = lax.axis_index("c")
    sid = lax.axis_index("s")
    wid = sid * _NC + cid

    pltpu.sync_copy(z_hbm, acc.at[pl.ds(sid * _RPT, _RPT)])
    plsc.subcore_barrier()

    def body(j, carry):
        base = wid * _EPT + j * _C
        pltpu.sync_copy(src_hbm.at[pl.ds(base, _C)], sidx)
        pltpu.sync_copy(dst_hbm.at[pl.ds(base, _C)], didx)
        pltpu.async_copy(h_hbm.at[sidx], rows, sem).wait()
        pltpu.sync_copy(rows, acc.at[didx], add=True)
        return carry

    lax.fori_loop(0, _NCHUNK, body, 0)
    plsc.subcore_barrier()
    pltpu.sync_copy(
        acc.at[pl.ds(sid * _RPT, _RPT)],
        out_hbm.at[pl.ds(cid * _NROWS + sid * _RPT, _RPT)],
    )


def kernel(A_a, X_a, Wr, br, W1, b1, W2, b2):
    del Wr, br
    pad = _EPAD - _E
    src = jnp.concatenate([A_a[0], jnp.zeros((pad,), jnp.int32)])
    dst = jnp.concatenate([A_a[1], jnp.full((pad,), _N, jnp.int32)])
    zrows = jnp.zeros((_RPT, _D), jnp.float32)

    h1 = _mm_bias(X_a, W1, b1.reshape(1, _D))
    p = _aggregate(h1, src, dst, zrows)
    h2 = _merge_relu_mm(p[:_N], p[_NROWS:_NROWS + _N], W2, b2.reshape(1, _D))
    q = _aggregate(h2, src, dst, zrows)
    return _merge_add(q[:_N], q[_NROWS:_NROWS + _N])
